# baseline (speedup 1.0000x reference)
import functools

import jax
import jax.numpy as jnp
from jax import lax
from jax.experimental import pallas as pl
from jax.experimental.pallas import tpu as pltpu
from jax.experimental.pallas import tpu_sc as plsc

_NUM_CORES = 2
_NUM_SUBCORES = 16
_NW = _NUM_CORES * _NUM_SUBCORES
_CHUNK = 80


@functools.partial(jax.jit, static_argnums=(3, 4, 5))
def _scatter_partials(X, a3, b3, N_pad, D, n_chunks):
    rows_per_tile = N_pad // _NUM_SUBCORES
    n_zfull = rows_per_tile // _CHUNK
    zrem = rows_per_tile - n_zfull * _CHUNK
    n_pairs = n_chunks // 2
    mesh = plsc.VectorSubcoreMesh(core_axis_name="c", subcore_axis_name="s")

    @functools.partial(
        pl.kernel,
        out_type=jax.ShapeDtypeStruct((_NUM_CORES, N_pad, D), jnp.float32),
        mesh=mesh,
        scratch_types=[
            pltpu.VMEM((n_chunks // 4, _CHUNK), jnp.int32),
            pltpu.VMEM((n_chunks // 4, _CHUNK), jnp.int32),
            pltpu.VMEM((_CHUNK, D), jnp.float32),
            pltpu.VMEM((_CHUNK, D), jnp.float32),
            pltpu.VMEM((_CHUNK, D), jnp.float32),
            pltpu.VMEM((_CHUNK, D), jnp.float32),
            pltpu.VMEM_SHARED((N_pad, D), jnp.float32),
            pltpu.SemaphoreType.DMA,
            pltpu.SemaphoreType.DMA,
            pltpu.SemaphoreType.DMA,
        ],
    )
    def sc_kernel(x_hbm, a_hbm, b_hbm, out_hbm, idx_a, idx_b, rows0, rows1,
                  rows2, rows3, acc, sem_i, sem_g, sem_s):
        c = lax.axis_index("c")
        s = lax.axis_index("s")
        wid = s * _NUM_CORES + c
        rows = (rows0, rows1, rows2, rows3)
        nch_t = n_chunks // 4

        d_ia = pltpu.async_copy(a_hbm.at[wid, 0], idx_a, sem_i)
        d_ib = pltpu.async_copy(b_hbm.at[wid, 0], idx_b, sem_i)

        zero = jnp.zeros((16,), jnp.float32)

        def zero_row(r, _):
            for j in range(D // 16):
                rows3[r, pl.ds(j * 16, 16)] = zero
            return 0

        lax.fori_loop(0, _CHUNK, zero_row, 0)

        d_ia.wait()
        d_ib.wait()
        pltpu.async_copy(x_hbm.at[idx_a.at[0]], rows0, sem_g)
        pltpu.async_copy(x_hbm.at[idx_a.at[1]], rows1, sem_g)

        base = s * rows_per_tile
        for k in range(n_zfull):
            pltpu.sync_copy(rows3, acc.at[pl.ds(base + k * _CHUNK, _CHUNK)])
        if zrem:
            pltpu.sync_copy(rows3.at[pl.ds(0, zrem)],
                            acc.at[pl.ds(base + n_zfull * _CHUNK, zrem)])

        plsc.subcore_barrier()

        def wait_gather(buf):
            pltpu.make_async_copy(x_hbm.at[idx_a.at[0]], buf, sem_g).wait()

        def wait_scatter():
            pltpu.make_async_copy(rows0, acc.at[idx_b.at[0]], sem_s).wait()

        def make_quad(first_quarter):
            def quad_body(t, _):
                for q in range(4):
                    j = 4 * t + q
                    wait_gather(rows[q])
                    pltpu.async_copy(rows[q], acc.at[idx_b.at[j]], sem_s,
                                     add=True)
                    if first_quarter and q < 2:
                        @pl.when(t > 0)
                        def _():
                            wait_scatter()
                    else:
                        wait_scatter()
                    nxt = jnp.minimum(j + 2, nch_t - 1)
                    pltpu.async_copy(x_hbm.at[idx_a.at[nxt]],
                                     rows[(q + 2) % 4], sem_g)
                return 0
            return quad_body

        for h in range(4):
            if h > 0:
                wait_gather(rows0)
                wait_gather(rows1)
                pltpu.sync_copy(a_hbm.at[wid, h], idx_a)
                pltpu.sync_copy(b_hbm.at[wid, h], idx_b)
                pltpu.async_copy(x_hbm.at[idx_a.at[0]], rows0, sem_g)
                pltpu.async_copy(x_hbm.at[idx_a.at[1]], rows1, sem_g)
            lax.fori_loop(0, nch_t // 4, make_quad(h == 0), 0)

        wait_gather(rows0)
        wait_gather(rows1)
        wait_scatter()
        wait_scatter()

        plsc.subcore_barrier()

        pltpu.sync_copy(
            acc.at[pl.ds(base, rows_per_tile)],
            out_hbm.at[c, pl.ds(base, rows_per_tile)],
        )

    return sc_kernel(X, a3, b3)


def _mlp(partials, N, W_hidden, W_out, b_hidden, b_out):
    _, N_pad, D = partials.shape
    U = W_out.shape[1]
    blk = 2000

    def tc_kernel(p_ref, wh_ref, wo_ref, bh_ref, bo_ref, o_ref):
        agg = p_ref[0] + p_ref[1]
        hid = jnp.dot(agg, wh_ref[...], preferred_element_type=jnp.float32)
        hid = hid + bh_ref[...]
        out = jnp.dot(hid, wo_ref[...], preferred_element_type=jnp.float32)
        o_ref[...] = out + bo_ref[...]

    return pl.pallas_call(
        tc_kernel,
        grid=(N // blk,),
        in_specs=[
            pl.BlockSpec((2, blk, D), lambda i: (0, i, 0)),
            pl.BlockSpec((D, W_hidden.shape[1]), lambda i: (0, 0)),
            pl.BlockSpec((W_out.shape[0], U), lambda i: (0, 0)),
            pl.BlockSpec((1, W_hidden.shape[1]), lambda i: (0, 0)),
            pl.BlockSpec((1, U), lambda i: (0, 0)),
        ],
        out_specs=pl.BlockSpec((blk, U), lambda i: (i, 0)),
        out_shape=jax.ShapeDtypeStruct((N, U), jnp.float32),
    )(partials, W_hidden, W_out, b_hidden.reshape(1, -1), b_out.reshape(1, -1))


def kernel(X, ref_a, ref_b, W_hidden, W_out, b_hidden, b_out):
    N, D = X.shape
    E = ref_a.shape[0]
    unit = _NW * _CHUNK * 16
    E_pad = -(-E // unit) * unit
    e_per_w = E_pad // _NW
    n_chunks = e_per_w // _CHUNK
    n_pad = -(-(N + 1) // (_NUM_SUBCORES * 8)) * (_NUM_SUBCORES * 8)
    a_i = ref_a.astype(jnp.int32)
    b_i = ref_b.astype(jnp.int32)
    if E_pad != E:
        pad = E_pad - E
        pad_ids = jnp.arange(pad, dtype=jnp.int32)
        a_i = jnp.concatenate([a_i, pad_ids % N])
        b_i = jnp.concatenate([b_i, N + pad_ids % (n_pad - N)])
    a4 = a_i.reshape(_NW, 4, n_chunks // 4, _CHUNK)
    b4 = b_i.reshape(_NW, 4, n_chunks // 4, _CHUNK)
    partials = _scatter_partials(X, a4, b4, n_pad, D, n_chunks)
    return _mlp(partials, N, W_hidden, W_out, b_hidden, b_out)


# speedup vs baseline: 1.0556x; 1.0556x over previous
import functools

import jax
import jax.numpy as jnp
from jax import lax
from jax.experimental import pallas as pl
from jax.experimental.pallas import tpu as pltpu
from jax.experimental.pallas import tpu_sc as plsc

_NUM_CORES = 2
_NUM_SUBCORES = 16
_NW = _NUM_CORES * _NUM_SUBCORES
_CHUNK = 80


@functools.partial(jax.jit, static_argnums=(3, 4, 5))
def _scatter_partials(X, a3, b3, N_pad, D, n_chunks):
    rows_per_tile = N_pad // _NUM_SUBCORES
    n_zfull = rows_per_tile // _CHUNK
    zrem = rows_per_tile - n_zfull * _CHUNK
    n_pairs = n_chunks // 2
    mesh = plsc.VectorSubcoreMesh(core_axis_name="c", subcore_axis_name="s")

    @functools.partial(
        pl.kernel,
        out_type=jax.ShapeDtypeStruct((_NUM_CORES, N_pad, D), jnp.float32),
        mesh=mesh,
        scratch_types=[
            pltpu.VMEM((n_chunks // 3, _CHUNK), jnp.int32),
            pltpu.VMEM((n_chunks // 3, _CHUNK), jnp.int32),
            pltpu.VMEM((_CHUNK, D), jnp.float32),
            pltpu.VMEM((_CHUNK, D), jnp.float32),
            pltpu.VMEM((_CHUNK, D), jnp.float32),
            pltpu.VMEM_SHARED((N_pad, D), jnp.float32),
            pltpu.SemaphoreType.DMA,
            pltpu.SemaphoreType.DMA,
            pltpu.SemaphoreType.DMA,
        ],
    )
    def sc_kernel(x_hbm, a_hbm, b_hbm, out_hbm, idx_a, idx_b, rows0, rows1,
                  rows2, acc, sem_i, sem_g, sem_s):
        c = lax.axis_index("c")
        s = lax.axis_index("s")
        wid = s * _NUM_CORES + c
        rows = (rows0, rows1, rows2)
        nch_t = n_chunks // 3

        d_ia = pltpu.async_copy(a_hbm.at[wid, 0], idx_a, sem_i)
        d_ib = pltpu.async_copy(b_hbm.at[wid, 0], idx_b, sem_i)

        zero = jnp.zeros((16,), jnp.float32)

        def zero_row(r, _):
            for j in range(D // 16):
                rows2[r, pl.ds(j * 16, 16)] = zero
            return 0

        lax.fori_loop(0, _CHUNK, zero_row, 0)

        d_ia.wait()
        d_ib.wait()
        pltpu.async_copy(x_hbm.at[idx_a.at[0]], rows0, sem_g)
        pltpu.async_copy(x_hbm.at[idx_a.at[1]], rows1, sem_g)

        base = s * rows_per_tile
        for k in range(n_zfull):
            pltpu.sync_copy(rows2, acc.at[pl.ds(base + k * _CHUNK, _CHUNK)])
        if zrem:
            pltpu.sync_copy(rows2.at[pl.ds(0, zrem)],
                            acc.at[pl.ds(base + n_zfull * _CHUNK, zrem)])

        plsc.subcore_barrier()

        def wait_gather(buf):
            pltpu.make_async_copy(x_hbm.at[idx_a.at[0]], buf, sem_g).wait()

        def wait_scatter():
            pltpu.make_async_copy(rows0, acc.at[idx_b.at[0]], sem_s).wait()

        def make_triple(first_third):
            def triple_body(t, _):
                for q in range(3):
                    j = 3 * t + q
                    wait_gather(rows[q])
                    pltpu.async_copy(rows[q], acc.at[idx_b.at[j]], sem_s,
                                     add=True)
                    if first_third and q == 0:
                        @pl.when(t > 0)
                        def _():
                            wait_scatter()
                    else:
                        wait_scatter()
                    nxt = jnp.minimum(j + 2, nch_t - 1)
                    pltpu.async_copy(x_hbm.at[idx_a.at[nxt]],
                                     rows[(q + 2) % 3], sem_g)
                return 0
            return triple_body

        for h in range(3):
            if h > 0:
                wait_gather(rows0)
                wait_gather(rows1)
                pltpu.sync_copy(a_hbm.at[wid, h], idx_a)
                pltpu.sync_copy(b_hbm.at[wid, h], idx_b)
                pltpu.async_copy(x_hbm.at[idx_a.at[0]], rows0, sem_g)
                pltpu.async_copy(x_hbm.at[idx_a.at[1]], rows1, sem_g)
            lax.fori_loop(0, nch_t // 3, make_triple(h == 0), 0)

        wait_gather(rows0)
        wait_gather(rows1)
        wait_scatter()

        plsc.subcore_barrier()

        pltpu.sync_copy(
            acc.at[pl.ds(base, rows_per_tile)],
            out_hbm.at[c, pl.ds(base, rows_per_tile)],
        )

    return sc_kernel(X, a3, b3)


def _mlp(partials, N, W_hidden, W_out, b_hidden, b_out):
    _, N_pad, D = partials.shape
    U = W_out.shape[1]
    blk = 5000

    def tc_kernel(p_ref, wh_ref, wo_ref, bh_ref, bo_ref, o_ref):
        agg = p_ref[0] + p_ref[1]
        hid = jnp.dot(agg, wh_ref[...], preferred_element_type=jnp.float32)
        hid = hid + bh_ref[...]
        out = jnp.dot(hid, wo_ref[...], preferred_element_type=jnp.float32)
        o_ref[...] = out + bo_ref[...]

    return pl.pallas_call(
        tc_kernel,
        grid=(N // blk,),
        in_specs=[
            pl.BlockSpec((2, blk, D), lambda i: (0, i, 0)),
            pl.BlockSpec((D, W_hidden.shape[1]), lambda i: (0, 0)),
            pl.BlockSpec((W_out.shape[0], U), lambda i: (0, 0)),
            pl.BlockSpec((1, W_hidden.shape[1]), lambda i: (0, 0)),
            pl.BlockSpec((1, U), lambda i: (0, 0)),
        ],
        out_specs=pl.BlockSpec((blk, U), lambda i: (i, 0)),
        out_shape=jax.ShapeDtypeStruct((N, U), jnp.float32),
    )(partials, W_hidden, W_out, b_hidden.reshape(1, -1), b_out.reshape(1, -1))


def kernel(X, ref_a, ref_b, W_hidden, W_out, b_hidden, b_out):
    N, D = X.shape
    E = ref_a.shape[0]
    unit = _NW * _CHUNK * 9
    E_pad = -(-E // unit) * unit
    e_per_w = E_pad // _NW
    n_chunks = e_per_w // _CHUNK
    n_pad = -(-(N + 1) // (_NUM_SUBCORES * 8)) * (_NUM_SUBCORES * 8)
    a_i = ref_a.astype(jnp.int32)
    b_i = ref_b.astype(jnp.int32)
    if E_pad != E:
        pad = E_pad - E
        pad_ids = jnp.arange(pad, dtype=jnp.int32)
        a_i = jnp.concatenate([a_i, pad_ids % N])
        b_i = jnp.concatenate([b_i, N + pad_ids % (n_pad - N)])
    a4 = a_i.reshape(_NW, 3, n_chunks // 3, _CHUNK)
    b4 = b_i.reshape(_NW, 3, n_chunks // 3, _CHUNK)
    partials = _scatter_partials(X, a4, b4, n_pad, D, n_chunks)
    return _mlp(partials, N, W_hidden, W_out, b_hidden, b_out)


# speedup vs baseline: 1.0825x; 1.0255x over previous
import functools

import jax
import jax.numpy as jnp
from jax import lax
from jax.experimental import pallas as pl
from jax.experimental.pallas import tpu as pltpu
from jax.experimental.pallas import tpu_sc as plsc

_NUM_CORES = 2
_NUM_SUBCORES = 16
_NW = _NUM_CORES * _NUM_SUBCORES
_CHUNK = 96


@functools.partial(jax.jit, static_argnums=(3, 4, 5))
def _scatter_partials(X, a3, b3, N_pad, D, n_chunks):
    rows_per_tile = N_pad // _NUM_SUBCORES
    n_zfull = rows_per_tile // _CHUNK
    zrem = rows_per_tile - n_zfull * _CHUNK
    n_pairs = n_chunks // 2
    mesh = plsc.VectorSubcoreMesh(core_axis_name="c", subcore_axis_name="s")

    @functools.partial(
        pl.kernel,
        out_type=jax.ShapeDtypeStruct((_NUM_CORES, N_pad, D), jnp.float32),
        mesh=mesh,
        scratch_types=[
            pltpu.VMEM((n_chunks // 3, _CHUNK), jnp.int32),
            pltpu.VMEM((n_chunks // 3, _CHUNK), jnp.int32),
            pltpu.VMEM((_CHUNK, D), jnp.float32),
            pltpu.VMEM((_CHUNK, D), jnp.float32),
            pltpu.VMEM((_CHUNK, D), jnp.float32),
            pltpu.VMEM_SHARED((N_pad, D), jnp.float32),
            pltpu.SemaphoreType.DMA,
            pltpu.SemaphoreType.DMA,
            pltpu.SemaphoreType.DMA,
        ],
    )
    def sc_kernel(x_hbm, a_hbm, b_hbm, out_hbm, idx_a, idx_b, rows0, rows1,
                  rows2, acc, sem_i, sem_g, sem_s):
        c = lax.axis_index("c")
        s = lax.axis_index("s")
        wid = s * _NUM_CORES + c
        rows = (rows0, rows1, rows2)
        nch_t = n_chunks // 3

        d_ia = pltpu.async_copy(a_hbm.at[wid, 0], idx_a, sem_i)
        d_ib = pltpu.async_copy(b_hbm.at[wid, 0], idx_b, sem_i)

        zero = jnp.zeros((16,), jnp.float32)

        def zero_row(r, _):
            for j in range(D // 16):
                rows2[r, pl.ds(j * 16, 16)] = zero
            return 0

        lax.fori_loop(0, _CHUNK, zero_row, 0)

        d_ia.wait()
        d_ib.wait()
        pltpu.async_copy(x_hbm.at[idx_a.at[0]], rows0, sem_g)
        pltpu.async_copy(x_hbm.at[idx_a.at[1]], rows1, sem_g)

        base = s * rows_per_tile
        for k in range(n_zfull):
            pltpu.sync_copy(rows2, acc.at[pl.ds(base + k * _CHUNK, _CHUNK)])
        if zrem:
            pltpu.sync_copy(rows2.at[pl.ds(0, zrem)],
                            acc.at[pl.ds(base + n_zfull * _CHUNK, zrem)])

        plsc.subcore_barrier()

        def wait_gather(buf):
            pltpu.make_async_copy(x_hbm.at[idx_a.at[0]], buf, sem_g).wait()

        def wait_scatter():
            pltpu.make_async_copy(rows0, acc.at[idx_b.at[0]], sem_s).wait()

        def make_triple(first_third):
            def triple_body(t, _):
                for q in range(3):
                    j = 3 * t + q
                    wait_gather(rows[q])
                    pltpu.async_copy(rows[q], acc.at[idx_b.at[j]], sem_s,
                                     add=True)
                    if first_third and q == 0:
                        @pl.when(t > 0)
                        def _():
                            wait_scatter()
                    else:
                        wait_scatter()
                    nxt = jnp.minimum(j + 2, nch_t - 1)
                    pltpu.async_copy(x_hbm.at[idx_a.at[nxt]],
                                     rows[(q + 2) % 3], sem_g)
                return 0
            return triple_body

        for h in range(3):
            if h > 0:
                wait_gather(rows0)
                wait_gather(rows1)
                pltpu.sync_copy(a_hbm.at[wid, h], idx_a)
                pltpu.sync_copy(b_hbm.at[wid, h], idx_b)
                pltpu.async_copy(x_hbm.at[idx_a.at[0]], rows0, sem_g)
                pltpu.async_copy(x_hbm.at[idx_a.at[1]], rows1, sem_g)
            lax.fori_loop(0, nch_t // 3, make_triple(h == 0), 0)

        wait_gather(rows0)
        wait_gather(rows1)
        wait_scatter()

        plsc.subcore_barrier()

        pltpu.sync_copy(
            acc.at[pl.ds(base, rows_per_tile)],
            out_hbm.at[c, pl.ds(base, rows_per_tile)],
        )

    return sc_kernel(X, a3, b3)


def _mlp(partials, N, W_hidden, W_out, b_hidden, b_out):
    _, N_pad, D = partials.shape
    U = W_out.shape[1]
    blk = 5000

    def tc_kernel(p_ref, wh_ref, wo_ref, bh_ref, bo_ref, o_ref):
        agg = p_ref[0] + p_ref[1]
        hid = jnp.dot(agg, wh_ref[...], preferred_element_type=jnp.float32)
        hid = hid + bh_ref[...]
        out = jnp.dot(hid, wo_ref[...], preferred_element_type=jnp.float32)
        o_ref[...] = out + bo_ref[...]

    return pl.pallas_call(
        tc_kernel,
        grid=(N // blk,),
        in_specs=[
            pl.BlockSpec((2, blk, D), lambda i: (0, i, 0)),
            pl.BlockSpec((D, W_hidden.shape[1]), lambda i: (0, 0)),
            pl.BlockSpec((W_out.shape[0], U), lambda i: (0, 0)),
            pl.BlockSpec((1, W_hidden.shape[1]), lambda i: (0, 0)),
            pl.BlockSpec((1, U), lambda i: (0, 0)),
        ],
        out_specs=pl.BlockSpec((blk, U), lambda i: (i, 0)),
        out_shape=jax.ShapeDtypeStruct((N, U), jnp.float32),
    )(partials, W_hidden, W_out, b_hidden.reshape(1, -1), b_out.reshape(1, -1))


def kernel(X, ref_a, ref_b, W_hidden, W_out, b_hidden, b_out):
    N, D = X.shape
    E = ref_a.shape[0]
    unit = _NW * _CHUNK * 9
    E_pad = -(-E // unit) * unit
    e_per_w = E_pad // _NW
    n_chunks = e_per_w // _CHUNK
    n_pad = -(-(N + 1) // (_NUM_SUBCORES * 8)) * (_NUM_SUBCORES * 8)
    a_i = ref_a.astype(jnp.int32)
    b_i = ref_b.astype(jnp.int32)
    if E_pad != E:
        pad = E_pad - E
        pad_ids = jnp.arange(pad, dtype=jnp.int32)
        a_i = jnp.concatenate([a_i, pad_ids % N])
        b_i = jnp.concatenate([b_i, N + pad_ids % (n_pad - N)])
    a4 = a_i.reshape(_NW, 3, n_chunks // 3, _CHUNK)
    b4 = b_i.reshape(_NW, 3, n_chunks // 3, _CHUNK)
    partials = _scatter_partials(X, a4, b4, n_pad, D, n_chunks)
    return _mlp(partials, N, W_hidden, W_out, b_hidden, b_out)


# speedup vs baseline: 1.1116x; 1.0269x over previous
import functools

import jax
import jax.numpy as jnp
from jax import lax
from jax.experimental import pallas as pl
from jax.experimental.pallas import tpu as pltpu
from jax.experimental.pallas import tpu_sc as plsc

_NUM_CORES = 2
_NUM_SUBCORES = 16
_NW = _NUM_CORES * _NUM_SUBCORES
_CHUNK = 96


@functools.partial(jax.jit, static_argnums=(3, 4, 5))
def _scatter_partials(X, a3, b3, N_pad, D, n_chunks):
    rows_per_tile = N_pad // _NUM_SUBCORES
    n_zfull = rows_per_tile // _CHUNK
    zrem = rows_per_tile - n_zfull * _CHUNK
    n_pairs = n_chunks // 2
    mesh = plsc.VectorSubcoreMesh(core_axis_name="c", subcore_axis_name="s")

    @functools.partial(
        pl.kernel,
        out_type=jax.ShapeDtypeStruct((_NUM_CORES, N_pad, D), jnp.float32),
        mesh=mesh,
        scratch_types=[
            pltpu.VMEM((n_chunks // 3, _CHUNK), jnp.int32),
            pltpu.VMEM((n_chunks // 3, _CHUNK), jnp.int32),
            pltpu.VMEM((_CHUNK, D), jnp.float32),
            pltpu.VMEM((_CHUNK, D), jnp.float32),
            pltpu.VMEM((_CHUNK, D), jnp.float32),
            pltpu.VMEM_SHARED((N_pad, D), jnp.float32),
            pltpu.SemaphoreType.DMA,
            pltpu.SemaphoreType.DMA,
            pltpu.SemaphoreType.DMA,
        ],
    )
    def sc_kernel(x_hbm, a_hbm, b_hbm, out_hbm, idx_a, idx_b, rows0, rows1,
                  rows2, acc, sem_i, sem_g, sem_s):
        c = lax.axis_index("c")
        s = lax.axis_index("s")
        wid = s * _NUM_CORES + c
        rows = (rows0, rows1, rows2)
        nch_t = n_chunks // 3

        d_ia = pltpu.async_copy(a_hbm.at[wid, 0], idx_a, sem_i)
        d_ib = pltpu.async_copy(b_hbm.at[wid, 0], idx_b, sem_i)

        zero = jnp.zeros((16,), jnp.float32)

        def zero_row(r, _):
            for j in range(D // 16):
                rows2[r, pl.ds(j * 16, 16)] = zero
            return 0

        lax.fori_loop(0, _CHUNK, zero_row, 0)

        d_ia.wait()
        d_ib.wait()
        pltpu.async_copy(x_hbm.at[idx_a.at[0]], rows0, sem_g)
        pltpu.async_copy(x_hbm.at[idx_a.at[1]], rows1, sem_g)

        base = s * rows_per_tile
        for k in range(n_zfull):
            pltpu.sync_copy(rows2, acc.at[pl.ds(base + k * _CHUNK, _CHUNK)])
        if zrem:
            pltpu.sync_copy(rows2.at[pl.ds(0, zrem)],
                            acc.at[pl.ds(base + n_zfull * _CHUNK, zrem)])

        plsc.subcore_barrier()

        def wait_gather(buf):
            pltpu.make_async_copy(x_hbm.at[idx_a.at[0]], buf, sem_g).wait()

        def wait_scatter():
            pltpu.make_async_copy(rows0, acc.at[idx_b.at[0]], sem_s).wait()

        def make_triple(first_third):
            def triple_body(t, _):
                for q in range(3):
                    j = 3 * t + q
                    wait_gather(rows[q])
                    pltpu.async_copy(rows[q], acc.at[idx_b.at[j]], sem_s,
                                     add=True)
                    if first_third and q == 0:
                        @pl.when(t > 0)
                        def _():
                            wait_scatter()
                    else:
                        wait_scatter()

                    @pl.when(j + 2 < nch_t)
                    def _():
                        pltpu.async_copy(x_hbm.at[idx_a.at[j + 2]],
                                         rows[(q + 2) % 3], sem_g)
                return 0
            return triple_body

        for h in range(3):
            if h > 0:
                pltpu.sync_copy(a_hbm.at[wid, h], idx_a)
                pltpu.sync_copy(b_hbm.at[wid, h], idx_b)
                pltpu.async_copy(x_hbm.at[idx_a.at[0]], rows0, sem_g)
                pltpu.async_copy(x_hbm.at[idx_a.at[1]], rows1, sem_g)
            lax.fori_loop(0, nch_t // 3, make_triple(h == 0), 0)

        wait_scatter()

        plsc.subcore_barrier()

        pltpu.sync_copy(
            acc.at[pl.ds(base, rows_per_tile)],
            out_hbm.at[c, pl.ds(base, rows_per_tile)],
        )

    return sc_kernel(X, a3, b3)


def _mlp(partials, N, W_hidden, W_out, b_hidden, b_out):
    _, N_pad, D = partials.shape
    U = W_out.shape[1]
    blk = 5000

    def tc_kernel(p_ref, wh_ref, wo_ref, bh_ref, bo_ref, o_ref):
        agg = p_ref[0] + p_ref[1]
        hid = jnp.dot(agg, wh_ref[...], preferred_element_type=jnp.float32)
        hid = hid + bh_ref[...]
        out = jnp.dot(hid, wo_ref[...], preferred_element_type=jnp.float32)
        o_ref[...] = out + bo_ref[...]

    return pl.pallas_call(
        tc_kernel,
        grid=(N // blk,),
        in_specs=[
            pl.BlockSpec((2, blk, D), lambda i: (0, i, 0)),
            pl.BlockSpec((D, W_hidden.shape[1]), lambda i: (0, 0)),
            pl.BlockSpec((W_out.shape[0], U), lambda i: (0, 0)),
            pl.BlockSpec((1, W_hidden.shape[1]), lambda i: (0, 0)),
            pl.BlockSpec((1, U), lambda i: (0, 0)),
        ],
        out_specs=pl.BlockSpec((blk, U), lambda i: (i, 0)),
        out_shape=jax.ShapeDtypeStruct((N, U), jnp.float32),
    )(partials, W_hidden, W_out, b_hidden.reshape(1, -1), b_out.reshape(1, -1))


def kernel(X, ref_a, ref_b, W_hidden, W_out, b_hidden, b_out):
    N, D = X.shape
    E = ref_a.shape[0]
    unit = _NW * _CHUNK * 9
    E_pad = -(-E // unit) * unit
    e_per_w = E_pad // _NW
    n_chunks = e_per_w // _CHUNK
    n_pad = -(-(N + 1) // (_NUM_SUBCORES * 8)) * (_NUM_SUBCORES * 8)
    a_i = ref_a.astype(jnp.int32)
    b_i = ref_b.astype(jnp.int32)
    if E_pad != E:
        pad = E_pad - E
        pad_ids = jnp.arange(pad, dtype=jnp.int32)
        a_i = jnp.concatenate([a_i, pad_ids % N])
        b_i = jnp.concatenate([b_i, N + pad_ids % (n_pad - N)])
    a4 = a_i.reshape(_NW, 3, n_chunks // 3, _CHUNK)
    b4 = b_i.reshape(_NW, 3, n_chunks // 3, _CHUNK)
    partials = _scatter_partials(X, a4, b4, n_pad, D, n_chunks)
    return _mlp(partials, N, W_hidden, W_out, b_hidden, b_out)
